# baseline (speedup 1.0000x reference)
import functools

import jax
import jax.numpy as jnp
from jax import lax
from jax.experimental import pallas as pl
from jax.experimental.pallas import tpu as pltpu
from jax.experimental.pallas import tpu_sc as plsc

H = 224
W = 224
C = 96
N = 4
P = H * W
B = N * P
ROW_CHUNK = 16

NUM_WORKERS = 32
ROWS_PER_W = (N * H) // NUM_WORKERS
HALF = 112
NHALF = ROWS_PER_W * 2


PAD = 128
RB = ROW_CHUNK * W


def _sum_body(x_ref, out_ref, pad_ref):
    j = pl.program_id(1)
    xb = x_ref[...]
    s = jnp.sum(xb, axis=(0, 1, 2)).reshape(1, 1, C)

    @pl.when(j == 0)
    def _():
        out_ref[...] = s

    @pl.when(j > 0)
    def _():
        out_ref[...] += s

    flat = xb.reshape(RB, C)
    pad_ref[...] = jnp.concatenate(
        [flat, jnp.zeros((RB, PAD - C), jnp.float32)], axis=1)


def _channel_sums(x):
    return pl.pallas_call(
        _sum_body,
        grid=(N, H // ROW_CHUNK),
        in_specs=[pl.BlockSpec((1, ROW_CHUNK, W, C), lambda n, j: (n, j, 0, 0))],
        out_specs=[
            pl.BlockSpec((1, 1, C), lambda n, j: (n, 0, 0)),
            pl.BlockSpec((RB, PAD), lambda n, j: (n * (H // ROW_CHUNK) + j, 0)),
        ],
        out_shape=[
            jax.ShapeDtypeStruct((N, 1, C), jnp.float32),
            jax.ShapeDtypeStruct((B, PAD), jnp.float32),
        ],
    )(x)


def _params_body(sums_ref, wloc_ref, bloc_ref, out_ref):
    mean = sums_ref[...].reshape(N, C) / float(P)
    mean_b = mean.astype(jnp.bfloat16)
    wloc_b = wloc_ref[...].astype(jnp.bfloat16)
    params = jnp.dot(mean_b, wloc_b,
                     preferred_element_type=jnp.float32) + bloc_ref[...]
    params = params.astype(jnp.bfloat16).astype(jnp.float32)
    padded = jnp.concatenate(
        [params, jnp.zeros((N, 10), jnp.float32)], axis=1)
    out_ref[...] = padded


def _params(sums, W_loc, b_loc):
    return pl.pallas_call(
        _params_body,
        grid=(1,),
        in_specs=[
            pl.BlockSpec((N, 1, C), lambda _: (0, 0, 0)),
            pl.BlockSpec((C, 6), lambda _: (0, 0)),
            pl.BlockSpec((1, 6), lambda _: (0, 0)),
        ],
        out_specs=pl.BlockSpec((N, 16), lambda _: (0, 0)),
        out_shape=jax.ShapeDtypeStruct((N, 16), jnp.float32),
    )(sums, W_loc, b_loc)


def _bf16_round(v):
    u = lax.bitcast_convert_type(v, jnp.uint32)
    r = (u + jnp.uint32(0x7FFF) + ((u >> jnp.uint32(16)) & jnp.uint32(1))) \
        & jnp.uint32(0xFFFF0000)
    return lax.bitcast_convert_type(r, jnp.float32)


def _floor(v):
    q = v.astype(jnp.int32).astype(jnp.float32)
    adj = jnp.where(q > v, jnp.float32(1.0), jnp.float32(0.0))
    f = q - adj
    return jnp.where(jnp.abs(v) < jnp.float32(8388608.0), f, v)


def _sc_gather_body(x_hbm, params_hbm, out_hbm,
                    pv, nxb_v, obt,
                    i0a, i1a, i2a, w0a, w1a, w2a, r0a, r1a, r2a,
                    i0b, i1b, i2b, w0b, w1b, w2b, r0b, r1b, r2b,
                    gsem_a, gsem_b, osem):
    wid = lax.axis_index("s") * 2 + lax.axis_index("c")
    n = wid // 8
    base_n = n * P
    row0 = wid * ROWS_PER_W

    pltpu.sync_copy(params_hbm.at[pl.ds(n * 16, 16)], pv)
    tv = pv[...]
    t0, t1, t2 = tv[0], tv[1], tv[2]
    t3, t4, t5 = tv[3], tv[4], tv[5]

    for g in range(W // 16):
        jx = (lax.iota(jnp.int32, 16) + g * 16).astype(jnp.float32)
        nxb_v[pl.ds(g * 16, 16)] = _bf16_round(2.0 * jx / 223.0 - 1.0)

    slots = (
        (i0a, i1a, i2a, w0a, w1a, w2a, r0a, r1a, r2a, gsem_a),
        (i0b, i1b, i2b, w0b, w1b, w2b, r0b, r1b, r2b, gsem_b),
    )

    def compute_idx(rl, half, slot):
        i0, i1, i2, w0, w1, w2 = slot[0:6]
        grow = row0 + rl
        irow = grow - n * H
        iv = jnp.full((16,), irow, jnp.int32).astype(jnp.float32)
        nyb = _bf16_round(2.0 * iv / 223.0 - 1.0)
        hy = t0 * nyb
        hx = t3 * nyb
        for g in range(HALF // 16):
            col = half * HALF + g * 16
            nxb = nxb_v[pl.ds(col, 16)]
            ty = (hy + t1 * nxb) + t2
            tx = (hx + t4 * nxb) + t5
            fy = _floor((ty + 1.0) * 223.0 / 2.0)
            fx = _floor((tx + 1.0) * 223.0 / 2.0)
            fy1 = fy + 1.0
            fx1 = fx + 1.0
            ay0 = 1.0 - jnp.abs(fy - ty)
            ay1 = 1.0 - jnp.abs(fy1 - ty)
            ax0 = 1.0 - jnp.abs(fx - tx)
            ax1 = 1.0 - jnp.abs(fx1 - tx)
            cy0 = jnp.clip(fy, 0.0, 223.0).astype(jnp.int32)
            cy1 = jnp.clip(fy1, 0.0, 223.0).astype(jnp.int32)
            cx0 = jnp.clip(fx, 0.0, 223.0).astype(jnp.int32)
            cx1 = jnp.clip(fx1, 0.0, 223.0).astype(jnp.int32)
            sl = pl.ds(g * 16, 16)
            i0[sl] = base_n + cy0 * W + cx0
            i1[sl] = base_n + cy1 * W + cx0
            i2[sl] = base_n + cy1 * W + cx1
            w0[sl] = ay0 * ax0
            w1[sl] = ay1 * ax0
            w2[sl] = 2.0 * ay1 * ax1

    def fire_gathers(slot):
        i0, i1, i2 = slot[0:3]
        r0, r1, r2 = slot[6:9]
        gsem = slot[9]
        pltpu.async_copy(x_hbm.at[i0], r0, gsem)
        pltpu.async_copy(x_hbm.at[i1], r1, gsem)
        pltpu.async_copy(x_hbm.at[i2], r2, gsem)

    def wait_gathers(slot):
        i0, i1, i2 = slot[0:3]
        r0, r1, r2 = slot[6:9]
        gsem = slot[9]
        pltpu.make_async_copy(x_hbm.at[i0], r0, gsem).wait()
        pltpu.make_async_copy(x_hbm.at[i1], r1, gsem).wait()
        pltpu.make_async_copy(x_hbm.at[i2], r2, gsem).wait()

    def out_slice(rl):
        return out_hbm.at[pl.ds((row0 + rl) * C, C)]

    crows = [lax.iota(jnp.int32, 16) + cc * 16 for cc in range(C // 16)]

    def combine(slot, col0):
        w0, w1, w2 = slot[3:6]
        r0, r1, r2 = slot[6:9]

        def q_body(q, carry):
            base16 = q * 16
            wv0 = w0[pl.ds(base16, 16)]
            wv1 = w1[pl.ds(base16, 16)]
            wv2 = w2[pl.ds(base16, 16)]
            for l in range(16):
                p = base16 + l
                a0 = wv0[l]
                a1 = wv1[l]
                a2 = wv2[l]
                cols = jnp.full((16,), col0 + p, jnp.int32)
                for cc in range(C // 16):
                    sl = pl.ds(cc * 16, 16)
                    v = a0 * r0[p, sl] + a1 * r1[p, sl] + a2 * r2[p, sl]
                    plsc.store_scatter(obt, [crows[cc], cols], v)
            return carry

        lax.fori_loop(0, HALF // 16, q_body, 0)

    for s in range(2):
        compute_idx(0, s, slots[s])
        fire_gathers(slots[s])

    def row_body(rl, carry):
        wait_gathers(slots[0])

        @pl.when(rl > 0)
        def _():
            pltpu.make_async_copy(obt, out_slice(rl), osem).wait()

        combine(slots[0], 0)

        @pl.when(rl < ROWS_PER_W - 1)
        def _():
            compute_idx(rl + 1, 0, slots[0])
            fire_gathers(slots[0])

        wait_gathers(slots[1])
        combine(slots[1], HALF)
        pltpu.async_copy(obt, out_slice(rl), osem)

        @pl.when(rl < ROWS_PER_W - 1)
        def _():
            compute_idx(rl + 1, 1, slots[1])
            fire_gathers(slots[1])

        return carry

    lax.fori_loop(0, ROWS_PER_W, row_body, 0)

    pltpu.make_async_copy(obt, out_slice(ROWS_PER_W - 1), osem).wait()


@functools.lru_cache(maxsize=None)
def _make_sc_gather():
  vm = pltpu.VMEM
  return pl.kernel(
    _sc_gather_body,
    out_type=jax.ShapeDtypeStruct((N * H * C, W), jnp.float32),
    mesh=plsc.VectorSubcoreMesh(core_axis_name="c", subcore_axis_name="s"),
    scratch_types=[vm((16,), jnp.float32), vm((W,), jnp.float32),
                   vm((C, W), jnp.float32)] + 2 * [
        vm((HALF,), jnp.int32), vm((HALF,), jnp.int32), vm((HALF,), jnp.int32),
        vm((HALF,), jnp.float32), vm((HALF,), jnp.float32), vm((HALF,), jnp.float32),
        vm((HALF, PAD), jnp.float32), vm((HALF, PAD), jnp.float32),
        vm((HALF, PAD), jnp.float32),
    ] + [pltpu.SemaphoreType.DMA] * 3,
    compiler_params=pltpu.CompilerParams(use_tc_tiling_on_sc=True,
                                         needs_layout_passes=False),
  )


def kernel(x, W_loc, b_loc):
    sums, x_pad = _channel_sums(x)
    params = _params(sums, W_loc, b_loc.reshape(1, 6))
    out_t = _make_sc_gather()(x_pad, params.reshape(N * 16))
    return out_t.reshape(N, H, C, W).transpose(0, 1, 3, 2)


# speedup vs baseline: 2.1745x; 2.1745x over previous
import functools

import jax
import jax.numpy as jnp
from jax import lax
from jax.experimental import pallas as pl
from jax.experimental.pallas import tpu as pltpu
from jax.experimental.pallas import tpu_sc as plsc

H = 224
W = 224
C = 96
N = 4
P = H * W
B = N * P
ROW_CHUNK = 16

NUM_WORKERS = 32
ROWS_PER_W = (N * H) // NUM_WORKERS
HALF = 112
NHALF = ROWS_PER_W * 2
OBW = W + 1


PAD = 128
RB = ROW_CHUNK * W


def _sum_body(xt_ref, out_ref, pad_ref):
    j = pl.program_id(1)
    xb = xt_ref[...]
    s = jnp.sum(xb, axis=(0, 1, 3)).reshape(1, 1, C)

    @pl.when(j == 0)
    def _():
        out_ref[...] = s

    @pl.when(j > 0)
    def _():
        out_ref[...] += s

    flat = jnp.transpose(xb, (0, 1, 3, 2)).reshape(RB, C)
    pad_ref[...] = jnp.concatenate(
        [flat, jnp.zeros((RB, PAD - C), jnp.float32)], axis=1)


def _channel_sums(xt):
    return pl.pallas_call(
        _sum_body,
        grid=(N, H // ROW_CHUNK),
        in_specs=[pl.BlockSpec((1, ROW_CHUNK, C, W), lambda n, j: (n, j, 0, 0))],
        out_specs=[
            pl.BlockSpec((1, 1, C), lambda n, j: (n, 0, 0)),
            pl.BlockSpec((RB, PAD), lambda n, j: (n * (H // ROW_CHUNK) + j, 0)),
        ],
        out_shape=[
            jax.ShapeDtypeStruct((N, 1, C), jnp.float32),
            jax.ShapeDtypeStruct((B, PAD), jnp.float32),
        ],
    )(xt)


def _params_body(sums_ref, wloc_ref, bloc_ref, out_ref):
    mean = sums_ref[...].reshape(N, C) / float(P)
    mean_b = mean.astype(jnp.bfloat16)
    wloc_b = wloc_ref[...].astype(jnp.bfloat16)
    params = jnp.dot(mean_b, wloc_b,
                     preferred_element_type=jnp.float32) + bloc_ref[...]
    params = params.astype(jnp.bfloat16).astype(jnp.float32)
    padded = jnp.concatenate(
        [params, jnp.zeros((N, 10), jnp.float32)], axis=1)
    out_ref[...] = padded


def _params(sums, W_loc, b_loc):
    return pl.pallas_call(
        _params_body,
        grid=(1,),
        in_specs=[
            pl.BlockSpec((N, 1, C), lambda _: (0, 0, 0)),
            pl.BlockSpec((C, 6), lambda _: (0, 0)),
            pl.BlockSpec((1, 6), lambda _: (0, 0)),
        ],
        out_specs=pl.BlockSpec((N, 16), lambda _: (0, 0)),
        out_shape=jax.ShapeDtypeStruct((N, 16), jnp.float32),
    )(sums, W_loc, b_loc)


def _bf16_round(v):
    u = lax.bitcast_convert_type(v, jnp.uint32)
    r = (u + jnp.uint32(0x7FFF) + ((u >> jnp.uint32(16)) & jnp.uint32(1))) \
        & jnp.uint32(0xFFFF0000)
    return lax.bitcast_convert_type(r, jnp.float32)


def _floor(v):
    q = v.astype(jnp.int32).astype(jnp.float32)
    adj = jnp.where(q > v, jnp.float32(1.0), jnp.float32(0.0))
    f = q - adj
    return jnp.where(jnp.abs(v) < jnp.float32(8388608.0), f, v)


def _sc_gather_body(x_hbm, params_hbm, out_hbm,
                    pv, nxb_v,
                    i0a, i1a, i2a, w0a, w1a, w2a, r0a, r1a, r2a, oba,
                    i0b, i1b, i2b, w0b, w1b, w2b, r0b, r1b, r2b, obb,
                    gsem_a, gsem_b, osem_a, osem_b):
    wid = lax.axis_index("s") * 2 + lax.axis_index("c")
    n = wid // 8
    base_n = n * P
    row0 = wid * ROWS_PER_W

    pltpu.sync_copy(params_hbm.at[pl.ds(n * 16, 16)], pv)
    tv = pv[...]
    t0, t1, t2 = tv[0], tv[1], tv[2]
    t3, t4, t5 = tv[3], tv[4], tv[5]

    for g in range(W // 16):
        jx = (lax.iota(jnp.int32, 16) + g * 16).astype(jnp.float32)
        nxb_v[pl.ds(g * 16, 16)] = _bf16_round(2.0 * jx / 223.0 - 1.0)

    slots = (
        (i0a, i1a, i2a, w0a, w1a, w2a, r0a, r1a, r2a, gsem_a, oba, osem_a),
        (i0b, i1b, i2b, w0b, w1b, w2b, r0b, r1b, r2b, gsem_b, obb, osem_b),
    )

    def compute_idx(rl, half, slot):
        i0, i1, i2, w0, w1, w2 = slot[0:6]
        grow = row0 + rl
        irow = grow - n * H
        iv = jnp.full((16,), irow, jnp.int32).astype(jnp.float32)
        nyb = _bf16_round(2.0 * iv / 223.0 - 1.0)
        hy = t0 * nyb
        hx = t3 * nyb
        for g in range(HALF // 16):
            col = half * HALF + g * 16
            nxb = nxb_v[pl.ds(col, 16)]
            ty = (hy + t1 * nxb) + t2
            tx = (hx + t4 * nxb) + t5
            fy = _floor((ty + 1.0) * 223.0 / 2.0)
            fx = _floor((tx + 1.0) * 223.0 / 2.0)
            fy1 = fy + 1.0
            fx1 = fx + 1.0
            ay0 = 1.0 - jnp.abs(fy - ty)
            ay1 = 1.0 - jnp.abs(fy1 - ty)
            ax0 = 1.0 - jnp.abs(fx - tx)
            ax1 = 1.0 - jnp.abs(fx1 - tx)
            cy0 = jnp.clip(fy, 0.0, 223.0).astype(jnp.int32)
            cy1 = jnp.clip(fy1, 0.0, 223.0).astype(jnp.int32)
            cx0 = jnp.clip(fx, 0.0, 223.0).astype(jnp.int32)
            cx1 = jnp.clip(fx1, 0.0, 223.0).astype(jnp.int32)
            sl = pl.ds(g * 16, 16)
            i0[sl] = base_n + cy0 * W + cx0
            i1[sl] = base_n + cy1 * W + cx0
            i2[sl] = base_n + cy1 * W + cx1
            w0[sl] = ay0 * ax0
            w1[sl] = ay1 * ax0
            w2[sl] = 2.0 * ay1 * ax1

    def fire_gathers(slot):
        i0, i1, i2 = slot[0:3]
        r0, r1, r2 = slot[6:9]
        gsem = slot[9]
        pltpu.async_copy(x_hbm.at[i0], r0, gsem)
        pltpu.async_copy(x_hbm.at[i1], r1, gsem)
        pltpu.async_copy(x_hbm.at[i2], r2, gsem)

    def wait_gathers(slot):
        i0, i1, i2 = slot[0:3]
        r0, r1, r2 = slot[6:9]
        gsem = slot[9]
        pltpu.make_async_copy(x_hbm.at[i0], r0, gsem).wait()
        pltpu.make_async_copy(x_hbm.at[i1], r1, gsem).wait()
        pltpu.make_async_copy(x_hbm.at[i2], r2, gsem).wait()

    def out_slice(rl, half):
        return out_hbm.at[pl.ds((row0 + rl) * W + half * HALF, HALF)]

    def combine(slot):
        w0, w1, w2 = slot[3:6]
        r0, r1, r2 = slot[6:9]
        ob = slot[10]

        def q_body(q, carry):
            base16 = q * 16
            wv0 = w0[pl.ds(base16, 16)]
            wv1 = w1[pl.ds(base16, 16)]
            wv2 = w2[pl.ds(base16, 16)]
            for l in range(16):
                p = base16 + l
                a0 = wv0[l]
                a1 = wv1[l]
                a2 = wv2[l]
                for cc in range(C // 16):
                    sl = pl.ds(cc * 16, 16)
                    ob[p, sl] = a0 * r0[p, sl] + a1 * r1[p, sl] + a2 * r2[p, sl]
            return carry

        lax.fori_loop(0, HALF // 16, q_body, 0)

    for s in range(2):
        compute_idx(0, s, slots[s])
        fire_gathers(slots[s])

    def row_body(rl, carry):
        for s in range(2):
            slot = slots[s]
            ob, osem = slot[10], slot[11]
            wait_gathers(slot)

            @pl.when(rl > 0)
            def _():
                pltpu.make_async_copy(ob, out_slice(rl, s), osem).wait()

            combine(slot)
            pltpu.async_copy(ob, out_slice(rl, s), osem)

            @pl.when(rl < ROWS_PER_W - 1)
            def _():
                compute_idx(rl + 1, s, slot)
                fire_gathers(slot)

        return carry

    lax.fori_loop(0, ROWS_PER_W, row_body, 0)

    for s in range(2):
        slot = slots[s]
        pltpu.make_async_copy(
            slot[10], out_slice(ROWS_PER_W - 1, s), slot[11]).wait()


@functools.lru_cache(maxsize=None)
def _make_sc_gather():
  vm = pltpu.VMEM
  return pl.kernel(
    _sc_gather_body,
    out_type=jax.ShapeDtypeStruct((B, C), jnp.float32),
    mesh=plsc.VectorSubcoreMesh(core_axis_name="c", subcore_axis_name="s"),
    scratch_types=[vm((16,), jnp.float32), vm((W,), jnp.float32)] + 2 * [
        vm((HALF,), jnp.int32), vm((HALF,), jnp.int32), vm((HALF,), jnp.int32),
        vm((HALF,), jnp.float32), vm((HALF,), jnp.float32), vm((HALF,), jnp.float32),
        vm((HALF, PAD), jnp.float32), vm((HALF, PAD), jnp.float32),
        vm((HALF, PAD), jnp.float32), vm((HALF, C), jnp.float32),
    ] + [pltpu.SemaphoreType.DMA] * 4,
    compiler_params=pltpu.CompilerParams(use_tc_tiling_on_sc=True),
  )


def kernel(x, W_loc, b_loc):
    xt = jnp.transpose(x, (0, 1, 3, 2))
    sums, x_pad = _channel_sums(xt)
    params = _params(sums, W_loc, b_loc.reshape(1, 6))
    out_flat = _make_sc_gather()(x_pad, params.reshape(N * 16))
    return out_flat.reshape(N, H, W, C)


# speedup vs baseline: 2.3518x; 1.0815x over previous
import functools

import jax
import jax.numpy as jnp
from jax import lax
from jax.experimental import pallas as pl
from jax.experimental.pallas import tpu as pltpu
from jax.experimental.pallas import tpu_sc as plsc

H = 224
W = 224
C = 96
N = 4
P = H * W
B = N * P
ROW_CHUNK = 16

NUM_WORKERS = 32
ROWS_PER_W = H // NUM_WORKERS
HALF = 112
PAD = 128
RB = ROW_CHUNK * W


def _sum_body(xt_ref, out_ref, pad_ref):
    j = pl.program_id(0)
    xb = xt_ref[...]
    s = jnp.sum(xb, axis=(0, 1, 3)).reshape(1, 1, C)

    @pl.when(j == 0)
    def _():
        out_ref[...] = s

    @pl.when(j > 0)
    def _():
        out_ref[...] += s

    flat = jnp.transpose(xb, (0, 1, 3, 2)).reshape(RB, C)
    pad_ref[...] = jnp.concatenate(
        [flat, jnp.zeros((RB, PAD - C), jnp.float32)], axis=1)


@functools.lru_cache(maxsize=None)
def _make_channel_sums(n):
    return pl.pallas_call(
        _sum_body,
        grid=(H // ROW_CHUNK,),
        in_specs=[pl.BlockSpec((1, ROW_CHUNK, C, W), lambda j: (n, j, 0, 0))],
        out_specs=[
            pl.BlockSpec((1, 1, C), lambda j: (0, 0, 0)),
            pl.BlockSpec((RB, PAD), lambda j: (j, 0)),
        ],
        out_shape=[
            jax.ShapeDtypeStruct((1, 1, C), jnp.float32),
            jax.ShapeDtypeStruct((P, PAD), jnp.float32),
        ],
    )


def _params_body(sums_ref, wloc_ref, bloc_ref, out_ref):
    mean = sums_ref[...].reshape(1, C) / float(P)
    mean_b = mean.astype(jnp.bfloat16)
    wloc_b = wloc_ref[...].astype(jnp.bfloat16)
    params = jnp.dot(mean_b, wloc_b,
                     preferred_element_type=jnp.float32) + bloc_ref[...]
    params = params.astype(jnp.bfloat16).astype(jnp.float32)
    out_ref[...] = jnp.concatenate(
        [params, jnp.zeros((1, 10), jnp.float32)], axis=1)


def _params(sums, W_loc, b_loc):
    return pl.pallas_call(
        _params_body,
        grid=(1,),
        in_specs=[
            pl.BlockSpec((1, 1, C), lambda _: (0, 0, 0)),
            pl.BlockSpec((C, 6), lambda _: (0, 0)),
            pl.BlockSpec((1, 6), lambda _: (0, 0)),
        ],
        out_specs=pl.BlockSpec((1, 16), lambda _: (0, 0)),
        out_shape=jax.ShapeDtypeStruct((1, 16), jnp.float32),
    )(sums, W_loc, b_loc)


def _bf16_round(v):
    u = lax.bitcast_convert_type(v, jnp.uint32)
    r = (u + jnp.uint32(0x7FFF) + ((u >> jnp.uint32(16)) & jnp.uint32(1))) \
        & jnp.uint32(0xFFFF0000)
    return lax.bitcast_convert_type(r, jnp.float32)


def _floor(v):
    q = v.astype(jnp.int32).astype(jnp.float32)
    adj = jnp.where(q > v, jnp.float32(1.0), jnp.float32(0.0))
    f = q - adj
    return jnp.where(jnp.abs(v) < jnp.float32(8388608.0), f, v)


def _sc_gather_body(x_hbm, params_hbm, out_hbm,
                    pv, nxb_v,
                    i0a, i1a, i2a, w0a, w1a, w2a, r0a, r1a, r2a, oba,
                    i0b, i1b, i2b, w0b, w1b, w2b, r0b, r1b, r2b, obb,
                    gsem_a, gsem_b, osem_a, osem_b):
    wid = lax.axis_index("s") * 2 + lax.axis_index("c")
    row0 = wid * ROWS_PER_W

    pltpu.sync_copy(params_hbm, pv)
    tv = pv[...]
    t0, t1, t2 = tv[0], tv[1], tv[2]
    t3, t4, t5 = tv[3], tv[4], tv[5]

    for g in range(W // 16):
        jx = (lax.iota(jnp.int32, 16) + g * 16).astype(jnp.float32)
        nxb_v[pl.ds(g * 16, 16)] = _bf16_round(2.0 * jx / 223.0 - 1.0)

    slots = (
        (i0a, i1a, i2a, w0a, w1a, w2a, r0a, r1a, r2a, gsem_a, oba, osem_a),
        (i0b, i1b, i2b, w0b, w1b, w2b, r0b, r1b, r2b, gsem_b, obb, osem_b),
    )

    def compute_idx(rl, half, slot):
        i0, i1, i2, w0, w1, w2 = slot[0:6]
        irow = row0 + rl
        iv = jnp.full((16,), irow, jnp.int32).astype(jnp.float32)
        nyb = _bf16_round(2.0 * iv / 223.0 - 1.0)
        hy = t0 * nyb
        hx = t3 * nyb
        for g in range(HALF // 16):
            col = half * HALF + g * 16
            nxb = nxb_v[pl.ds(col, 16)]
            ty = (hy + t1 * nxb) + t2
            tx = (hx + t4 * nxb) + t5
            fy = _floor((ty + 1.0) * 223.0 / 2.0)
            fx = _floor((tx + 1.0) * 223.0 / 2.0)
            fy1 = fy + 1.0
            fx1 = fx + 1.0
            ay0 = 1.0 - jnp.abs(fy - ty)
            ay1 = 1.0 - jnp.abs(fy1 - ty)
            ax0 = 1.0 - jnp.abs(fx - tx)
            ax1 = 1.0 - jnp.abs(fx1 - tx)
            cy0 = jnp.clip(fy, 0.0, 223.0).astype(jnp.int32)
            cy1 = jnp.clip(fy1, 0.0, 223.0).astype(jnp.int32)
            cx0 = jnp.clip(fx, 0.0, 223.0).astype(jnp.int32)
            cx1 = jnp.clip(fx1, 0.0, 223.0).astype(jnp.int32)
            sl = pl.ds(g * 16, 16)
            i0[sl] = cy0 * W + cx0
            i1[sl] = cy1 * W + cx0
            i2[sl] = cy1 * W + cx1
            w0[sl] = ay0 * ax0
            w1[sl] = ay1 * ax0
            w2[sl] = 2.0 * ay1 * ax1

    def fire_gathers(slot):
        i0, i1, i2 = slot[0:3]
        r0, r1, r2 = slot[6:9]
        gsem = slot[9]
        pltpu.async_copy(x_hbm.at[i0], r0, gsem)
        pltpu.async_copy(x_hbm.at[i1], r1, gsem)
        pltpu.async_copy(x_hbm.at[i2], r2, gsem)

    def wait_gathers(slot):
        i0, i1, i2 = slot[0:3]
        r0, r1, r2 = slot[6:9]
        gsem = slot[9]
        pltpu.make_async_copy(x_hbm.at[i0], r0, gsem).wait()
        pltpu.make_async_copy(x_hbm.at[i1], r1, gsem).wait()
        pltpu.make_async_copy(x_hbm.at[i2], r2, gsem).wait()

    def out_slice(rl, half):
        return out_hbm.at[pl.ds((row0 + rl) * W + half * HALF, HALF)]

    def combine(slot):
        w0, w1, w2 = slot[3:6]
        r0, r1, r2 = slot[6:9]
        ob = slot[10]

        def q_body(q, carry):
            base16 = q * 16
            wv0 = w0[pl.ds(base16, 16)]
            wv1 = w1[pl.ds(base16, 16)]
            wv2 = w2[pl.ds(base16, 16)]
            for l in range(16):
                p = base16 + l
                a0 = wv0[l]
                a1 = wv1[l]
                a2 = wv2[l]
                for cc in range(C // 16):
                    sl = pl.ds(cc * 16, 16)
                    ob[p, sl] = a0 * r0[p, sl] + a1 * r1[p, sl] + a2 * r2[p, sl]
            return carry

        lax.fori_loop(0, HALF // 16, q_body, 0)

    for s in range(2):
        compute_idx(0, s, slots[s])
        fire_gathers(slots[s])

    def row_body(rl, carry):
        for s in range(2):
            slot = slots[s]
            ob, osem = slot[10], slot[11]
            wait_gathers(slot)

            @pl.when(rl > 0)
            def _():
                pltpu.make_async_copy(ob, out_slice(rl, s), osem).wait()

            combine(slot)
            pltpu.async_copy(ob, out_slice(rl, s), osem)

            @pl.when(rl < ROWS_PER_W - 1)
            def _():
                compute_idx(rl + 1, s, slot)
                fire_gathers(slot)

        return carry

    lax.fori_loop(0, ROWS_PER_W, row_body, 0)

    for s in range(2):
        slot = slots[s]
        pltpu.make_async_copy(
            slot[10], out_slice(ROWS_PER_W - 1, s), slot[11]).wait()


@functools.lru_cache(maxsize=None)
def _make_sc_gather():
  vm = pltpu.VMEM
  return pl.kernel(
    _sc_gather_body,
    out_type=jax.ShapeDtypeStruct((P, C), jnp.float32),
    mesh=plsc.VectorSubcoreMesh(core_axis_name="c", subcore_axis_name="s"),
    scratch_types=[vm((16,), jnp.float32), vm((W,), jnp.float32)] + 2 * [
        vm((HALF,), jnp.int32), vm((HALF,), jnp.int32), vm((HALF,), jnp.int32),
        vm((HALF,), jnp.float32), vm((HALF,), jnp.float32), vm((HALF,), jnp.float32),
        vm((HALF, PAD), jnp.float32), vm((HALF, PAD), jnp.float32),
        vm((HALF, PAD), jnp.float32), vm((HALF, C), jnp.float32),
    ] + [pltpu.SemaphoreType.DMA] * 4,
    compiler_params=pltpu.CompilerParams(use_tc_tiling_on_sc=True),
  )


def _epi_body_first(src_ref, acc_ref):
    acc_ref[...] = jnp.transpose(src_ref[...], (0, 2, 1)).reshape(
        1, ROW_CHUNK, C, W)


def _epi_body(acc_in_ref, src_ref, acc_ref):
    del acc_in_ref
    acc_ref[...] = jnp.transpose(src_ref[...], (0, 2, 1)).reshape(
        1, ROW_CHUNK, C, W)


@functools.lru_cache(maxsize=None)
def _make_epi(n):
    out_spec = pl.BlockSpec((1, ROW_CHUNK, C, W), lambda j: (n, j, 0, 0))
    out_shape = jax.ShapeDtypeStruct((N, H, C, W), jnp.float32)
    src_spec = pl.BlockSpec((ROW_CHUNK, W, C), lambda j: (j, 0, 0))
    if n == 0:
        return pl.pallas_call(
            _epi_body_first,
            grid=(H // ROW_CHUNK,),
            in_specs=[src_spec],
            out_specs=out_spec,
            out_shape=out_shape,
        )
    return pl.pallas_call(
        _epi_body,
        grid=(H // ROW_CHUNK,),
        in_specs=[pl.BlockSpec(memory_space=pl.ANY), src_spec],
        out_specs=out_spec,
        out_shape=out_shape,
        input_output_aliases={0: 0},
    )


def kernel(x, W_loc, b_loc):
    xt = jnp.transpose(x, (0, 1, 3, 2))
    b2 = b_loc.reshape(1, 6)
    acc = None
    for n in range(N):
        sums, x_pad = _make_channel_sums(n)(xt)
        params = _params(sums, W_loc, b2)
        out_n = _make_sc_gather()(x_pad, params.reshape(16))
        src = out_n.reshape(H, W, C)
        if n == 0:
            acc = _make_epi(0)(src)
        else:
            acc = _make_epi(n)(acc, src)
    return jnp.transpose(acc, (0, 1, 3, 2))
